# baseline (speedup 1.0000x reference)
import dataclasses
import functools

import jax
import jax.numpy as jnp
from jax import lax
from jax.experimental import pallas as pl
from jax.experimental.pallas import tpu as pltpu
from jax.experimental.pallas import tpu_sc as plsc

_F32 = jnp.float32
_I32 = jnp.int32
_GRP = 128
_SB = 4
_NSUB = 16
_NCORE = 2
_NW = _NSUB * _NCORE
_D_GAT = 64
_D_RG = 32
_W = 128
_AWG = 80
_AWR = 48


def _sc_params():
    cp = pltpu.CompilerParams()
    if "needs_layout_passes" in pltpu.CompilerParams.__dataclass_fields__:
        cp = dataclasses.replace(cp, needs_layout_passes=False)
    return cp


def _row_split(n):
    per = ((n + _NSUB - 1) // _NSUB + 7) // 8 * 8
    last = n - per * (_NSUB - 1)
    assert last > 0 and last % 8 == 0
    return per, last


def _init_shared(z_hbm, s_sh, sid, n):
    per, last = _row_split(n)

    @pl.when(sid < _NSUB - 1)
    def _():
        pltpu.sync_copy(z_hbm.at[pl.ds(sid * per, per)],
                        s_sh.at[pl.ds(sid * per, per)])

    @pl.when(sid == _NSUB - 1)
    def _():
        pltpu.sync_copy(z_hbm.at[pl.ds(n - last, last)],
                        s_sh.at[pl.ds(n - last, last)])


def _dump_shared(s_sh, out_hbm, cid, sid, n):
    per, last = _row_split(n)

    @pl.when(sid < _NSUB - 1)
    def _():
        pltpu.sync_copy(s_sh.at[pl.ds(sid * per, per)],
                        out_hbm.at[cid, pl.ds(sid * per, per)])

    @pl.when(sid == _NSUB - 1)
    def _():
        pltpu.sync_copy(s_sh.at[pl.ds(n - last, last)],
                        out_hbm.at[cid, pl.ds(n - last, last)])


def _prep_body(x_ref, wg_ref, asw_ref, adw_ref,
               h_ref, as_ref, ad_ref, cv_ref, ps_ref):
    n = x_ref.shape[0]
    h = jnp.dot(x_ref[...], wg_ref[...], preferred_element_type=_F32)
    a_s = jnp.sum(h * asw_ref[...], axis=1, keepdims=True)
    a_d = jnp.sum(h * adw_ref[...], axis=1, keepdims=True)
    as_ref[...] = a_s
    ad_ref[...] = a_d
    cmax = jnp.max(a_s) + jnp.max(a_d)
    c = jnp.maximum(cmax, 0.2 * cmax)
    cv_ref[...] = jnp.full((1, 16), c, _F32)
    s = a_s + a_d
    e = jnp.maximum(s, 0.2 * s)
    ps_ref[...] = jnp.exp(e - c)
    h_ref[...] = jnp.concatenate(
        [h, jnp.ones((n, 1), _F32), jnp.zeros((n, _W - _D_GAT - 1), _F32)],
        axis=1)


def _prep(x, w_gat, att_src, att_dst):
    n = x.shape[0]
    return pl.pallas_call(
        _prep_body,
        out_shape=[
            jax.ShapeDtypeStruct((n, _W), _F32),
            jax.ShapeDtypeStruct((n, 1), _F32),
            jax.ShapeDtypeStruct((n, 1), _F32),
            jax.ShapeDtypeStruct((1, 16), _F32),
            jax.ShapeDtypeStruct((n, 1), _F32),
        ],
    )(x, w_gat, att_src.reshape(1, _D_GAT), att_dst.reshape(1, _D_GAT))


def _ppass_body(as_hbm, ad_hbm, cv_hbm, src_hbm, dst_hbm, p_out,
                as_v, ad_v, cv_v, sidx_v, didx_v, p_v, *, nsb, n):
    cid = lax.axis_index("c")
    sid = lax.axis_index("s")
    wid = sid * _NCORE + cid
    pltpu.sync_copy(as_hbm, as_v)
    pltpu.sync_copy(ad_hbm, ad_v)
    pltpu.sync_copy(cv_hbm, cv_v)
    cv = cv_v[...]
    nmy = nsb // _NW + jnp.where(wid < nsb % _NW, 1, 0)

    @pl.loop(0, nmy)
    def _(i):
        s = i * _NW + wid
        pltpu.sync_copy(src_hbm.at[s], sidx_v)
        pltpu.sync_copy(dst_hbm.at[s], didx_v)
        for jj in range(_SB * _GRP // 16):
            g, off = jj // (_GRP // 16), jj % (_GRP // 16) * 16
            si = sidx_v[g, pl.ds(off, 16)]
            di = didx_v[g, pl.ds(off, 16)]
            a = plsc.load_gather(as_v, [si]) + plsc.load_gather(ad_v, [di])
            e = jnp.maximum(a, 0.2 * a)
            p_v[g, pl.ds(off, 16)] = jnp.exp(e - cv)
        pltpu.sync_copy(p_v, p_out.at[s])


def _ppass(a_s, a_d, cvec, src3d, dst3d):
    n = a_s.shape[0]
    nsb = src3d.shape[0]
    mesh = plsc.VectorSubcoreMesh(core_axis_name="c", subcore_axis_name="s")
    return pl.kernel(
        functools.partial(_ppass_body, nsb=nsb, n=n),
        out_type=jax.ShapeDtypeStruct((nsb, _SB, _GRP), _F32),
        mesh=mesh,
        scratch_types=[
            pltpu.VMEM((n,), _F32),
            pltpu.VMEM((n,), _F32),
            pltpu.VMEM((16,), _F32),
            pltpu.VMEM((_SB, _GRP), _I32),
            pltpu.VMEM((_SB, _GRP), _I32),
            pltpu.VMEM((_SB, _GRP), _F32),
        ],
        compiler_params=_sc_params(),
    )(a_s, a_d, cvec, src3d, dst3d)


def _gat_body(h_hbm, p_hbm, src_hbm, dst_hbm, z_hbm, out_hbm,
              sidx_v, didx_v, p_v, rows_v, s_sh,
              sem_g0, sem_g1, sem_s0, sem_s1,
              *, nsb, n):
    cid = lax.axis_index("c")
    sid = lax.axis_index("s")
    wid = sid * _NCORE + cid
    _init_shared(z_hbm, s_sh, sid, n)
    plsc.subcore_barrier()
    nmy = nsb // _NW + jnp.where(wid < nsb % _NW, 1, 0)

    @pl.loop(0, nmy)
    def _(i):
        s = i * _NW + wid
        pltpu.sync_copy(src_hbm.at[s], sidx_v)
        pltpu.sync_copy(dst_hbm.at[s], didx_v)
        pltpu.sync_copy(p_hbm.at[s], p_v)
        sem_g = (sem_g0, sem_g1)
        sem_s = (sem_s0, sem_s1)
        gds = {0: pltpu.async_copy(h_hbm.at[sidx_v.at[0]], rows_v.at[0],
                                   sem_g[0])}
        sds = {}
        for g in range(_SB):
            b = g & 1
            gds[g].wait()

            @pl.loop(0, _GRP, unroll=4)
            def _(j):
                pj = plsc.load_gather(
                    p_v, [jnp.full((16,), g, _I32),
                          jnp.broadcast_to(j, (16,))])
                for k in range(_AWG // 16):
                    sl = (b, j, pl.ds(k * 16, 16))
                    rows_v[sl] = rows_v[sl] * pj

            if g + 1 < _SB:
                nb = (g + 1) & 1
                if nb in sds:
                    sds[nb].wait()
                gds[g + 1] = pltpu.async_copy(h_hbm.at[sidx_v.at[g + 1]],
                                              rows_v.at[nb], sem_g[nb])
            sds[b] = pltpu.async_copy(rows_v.at[b], s_sh.at[didx_v.at[g]],
                                      sem_s[b], add=True)
        for b in sds:
            sds[b].wait()

    plsc.subcore_barrier()
    _dump_shared(s_sh, out_hbm, cid, sid, n)


def _gat_pass(h128, p3d, src3d, dst3d):
    n = h128.shape[0]
    nsb = src3d.shape[0]
    mesh = plsc.VectorSubcoreMesh(core_axis_name="c", subcore_axis_name="s")
    return pl.kernel(
        functools.partial(_gat_body, nsb=nsb, n=n),
        out_type=jax.ShapeDtypeStruct((_NCORE, n, _W), _F32),
        mesh=mesh,
        scratch_types=[
            pltpu.VMEM((_SB, _GRP), _I32),
            pltpu.VMEM((_SB, _GRP), _I32),
            pltpu.VMEM((_SB, _GRP), _F32),
            pltpu.VMEM((2, _GRP, _W), _F32),
            pltpu.VMEM_SHARED((n, _W), _F32),
            pltpu.SemaphoreType.DMA,
            pltpu.SemaphoreType.DMA,
            pltpu.SemaphoreType.DMA,
            pltpu.SemaphoreType.DMA,
        ],
        compiler_params=_sc_params(),
    )(h128, p3d, src3d, dst3d, jnp.zeros((n, _W), _F32))


def _combine_body(s2_ref, h_ref, ps_ref, bg_ref, wrg_ref, x1_ref, y_ref):
    n = h_ref.shape[0]
    ssum = s2_ref[0] + s2_ref[1]
    ps = ps_ref[...]
    h = h_ref[:, :_D_GAT]
    num = ssum[:, :_D_GAT] + ps * h
    denom = ssum[:, _D_GAT:_D_GAT + 1] + ps
    gat = num / denom + bg_ref[...]
    x1 = jnp.maximum(gat, 0.0)
    x1_ref[...] = x1
    y = jnp.dot(x1, wrg_ref[...], preferred_element_type=_F32)
    y_ref[...] = jnp.concatenate(
        [y, jnp.ones((n, 1), _F32), jnp.zeros((n, _W - _D_RG - 1), _F32)],
        axis=1)


def _combine(s2, h128, p_self, b_gat, w_rgcn):
    n = h128.shape[0]
    return pl.pallas_call(
        _combine_body,
        out_shape=[
            jax.ShapeDtypeStruct((n, _D_GAT), _F32),
            jax.ShapeDtypeStruct((n, _W), _F32),
        ],
    )(s2, h128, p_self, b_gat.reshape(1, _D_GAT), w_rgcn)


def _rgcn_body(y_hbm, src_hbm, dst_hbm, z_hbm, out_hbm,
               sidx_v, didx_v, rows_v, s_sh,
               sem_g0, sem_g1, sem_s0, sem_s1,
               *, nsb, n):
    cid = lax.axis_index("c")
    sid = lax.axis_index("s")
    wid = sid * _NCORE + cid
    _init_shared(z_hbm, s_sh, sid, n)
    plsc.subcore_barrier()
    nmy = nsb // _NW + jnp.where(wid < nsb % _NW, 1, 0)

    @pl.loop(0, nmy)
    def _(i):
        s = i * _NW + wid
        pltpu.sync_copy(src_hbm.at[s], sidx_v)
        pltpu.sync_copy(dst_hbm.at[s], didx_v)
        sem_g = (sem_g0, sem_g1)
        sem_s = (sem_s0, sem_s1)
        gds = {0: pltpu.async_copy(y_hbm.at[sidx_v.at[0]], rows_v.at[0],
                                   sem_g[0])}
        sds = {}
        for g in range(_SB):
            b = g & 1
            gds[g].wait()
            if g + 1 < _SB:
                nb = (g + 1) & 1
                if nb in sds:
                    sds[nb].wait()
                gds[g + 1] = pltpu.async_copy(y_hbm.at[sidx_v.at[g + 1]],
                                              rows_v.at[nb], sem_g[nb])
            sds[b] = pltpu.async_copy(rows_v.at[b], s_sh.at[didx_v.at[g]],
                                      sem_s[b], add=True)
        for b in sds:
            sds[b].wait()

    plsc.subcore_barrier()
    _dump_shared(s_sh, out_hbm, cid, sid, n)


def _rgcn_pass(y128, src3d, dst3d):
    n = y128.shape[0]
    nsb = src3d.shape[0]
    mesh = plsc.VectorSubcoreMesh(core_axis_name="c", subcore_axis_name="s")
    return pl.kernel(
        functools.partial(_rgcn_body, nsb=nsb, n=n),
        out_type=jax.ShapeDtypeStruct((_NCORE, n, _W), _F32),
        mesh=mesh,
        scratch_types=[
            pltpu.VMEM((_SB, _GRP), _I32),
            pltpu.VMEM((_SB, _GRP), _I32),
            pltpu.VMEM((2, _GRP, _W), _F32),
            pltpu.VMEM_SHARED((n, _W), _F32),
            pltpu.SemaphoreType.DMA,
            pltpu.SemaphoreType.DMA,
            pltpu.SemaphoreType.DMA,
            pltpu.SemaphoreType.DMA,
        ],
        compiler_params=_sc_params(),
    )(y128, src3d, dst3d, jnp.zeros((n, _W), _F32))


def _final_body(a2_ref, x1_ref, wroot_ref, brg_ref, wfc_ref, bfc_ref,
                out_ref):
    asum = a2_ref[0] + a2_ref[1]
    cnt = jnp.maximum(asum[:, _D_RG:_D_RG + 1], 1.0)
    agg = asum[:, :_D_RG] / cnt
    x2 = (agg + jnp.dot(x1_ref[...], wroot_ref[...],
                        preferred_element_type=_F32) + brg_ref[...])
    out_ref[...] = (jnp.dot(x2, wfc_ref[...], preferred_element_type=_F32)
                    + bfc_ref[...])


def _final(a2, x1, w_root, b_rgcn, w_fc, b_fc):
    n = x1.shape[0]
    return pl.pallas_call(
        _final_body,
        out_shape=jax.ShapeDtypeStruct((n, w_fc.shape[1]), _F32),
    )(a2, x1, w_root, b_rgcn.reshape(1, _D_RG), w_fc,
      b_fc.reshape(1, w_fc.shape[1]))


def kernel(x, edge_index, W_gat, att_src, att_dst, b_gat, W_rgcn, W_root,
           b_rgcn, W_fc, b_fc):
    n = x.shape[0]
    n_edges = edge_index.shape[1]
    nsb = n_edges // (_SB * _GRP)
    src = edge_index[0].astype(_I32)
    dst = edge_index[1].astype(_I32)
    src3d = src.reshape(nsb, _SB, _GRP)
    dst3d = dst.reshape(nsb, _SB, _GRP)

    h128, a_s, a_d, cvec, p_self = _prep(x, W_gat, att_src, att_dst)
    p3d = _ppass(a_s.reshape(n), a_d.reshape(n), cvec.reshape(16),
                 src3d, dst3d)
    s2 = _gat_pass(h128, p3d, src3d, dst3d)
    x1, y128 = _combine(s2, h128, p_self, b_gat, W_rgcn)
    a2 = _rgcn_pass(y128, src3d, dst3d)
    return _final(a2, x1, W_root, b_rgcn, W_fc, b_fc)


# speedup vs baseline: 1.0918x; 1.0918x over previous
import dataclasses
import functools

import jax
import jax.numpy as jnp
from jax import lax
from jax.experimental import pallas as pl
from jax.experimental.pallas import tpu as pltpu
from jax.experimental.pallas import tpu_sc as plsc

_F32 = jnp.float32
_I32 = jnp.int32
_GRP = 128
_SB = 10
_NBUF = 2
_NSUB = 16
_NCORE = 2
_NW = _NSUB * _NCORE
_D_GAT = 64
_D_RG = 32
_W = 128
_AWG = 80
_AWR = 48


def _sc_params():
    cp = pltpu.CompilerParams()
    if "needs_layout_passes" in pltpu.CompilerParams.__dataclass_fields__:
        cp = dataclasses.replace(cp, needs_layout_passes=False)
    return cp


def _row_split(n):
    per = ((n + _NSUB - 1) // _NSUB + 7) // 8 * 8
    last = n - per * (_NSUB - 1)
    assert last > 0 and last % 8 == 0
    return per, last


def _init_shared(z_hbm, s_sh, sid, n):
    per, last = _row_split(n)

    @pl.when(sid < _NSUB - 1)
    def _():
        pltpu.sync_copy(z_hbm.at[pl.ds(sid * per, per)],
                        s_sh.at[pl.ds(sid * per, per)])

    @pl.when(sid == _NSUB - 1)
    def _():
        pltpu.sync_copy(z_hbm.at[pl.ds(n - last, last)],
                        s_sh.at[pl.ds(n - last, last)])


def _dump_shared(s_sh, out_hbm, cid, sid, n):
    per, last = _row_split(n)

    @pl.when(sid < _NSUB - 1)
    def _():
        pltpu.sync_copy(s_sh.at[pl.ds(sid * per, per)],
                        out_hbm.at[cid, pl.ds(sid * per, per)])

    @pl.when(sid == _NSUB - 1)
    def _():
        pltpu.sync_copy(s_sh.at[pl.ds(n - last, last)],
                        out_hbm.at[cid, pl.ds(n - last, last)])


def _prep_body(x_ref, wg_ref, asw_ref, adw_ref,
               h_ref, as_ref, ad_ref, cv_ref, ps_ref):
    n = x_ref.shape[0]
    h = jnp.dot(x_ref[...], wg_ref[...], preferred_element_type=_F32)
    a_s = jnp.sum(h * asw_ref[...], axis=1, keepdims=True)
    a_d = jnp.sum(h * adw_ref[...], axis=1, keepdims=True)
    as_ref[...] = a_s
    ad_ref[...] = a_d
    cmax = jnp.max(a_s) + jnp.max(a_d)
    c = jnp.maximum(cmax, 0.2 * cmax)
    cv_ref[...] = jnp.full((1, 16), c, _F32)
    s = a_s + a_d
    e = jnp.maximum(s, 0.2 * s)
    ps_ref[...] = jnp.exp(e - c)
    h_ref[...] = jnp.concatenate(
        [h, jnp.ones((n, 1), _F32), jnp.zeros((n, _W - _D_GAT - 1), _F32)],
        axis=1)


def _prep(x, w_gat, att_src, att_dst):
    n = x.shape[0]
    return pl.pallas_call(
        _prep_body,
        out_shape=[
            jax.ShapeDtypeStruct((n, _W), _F32),
            jax.ShapeDtypeStruct((n, 1), _F32),
            jax.ShapeDtypeStruct((n, 1), _F32),
            jax.ShapeDtypeStruct((1, 16), _F32),
            jax.ShapeDtypeStruct((n, 1), _F32),
        ],
    )(x, w_gat, att_src.reshape(1, _D_GAT), att_dst.reshape(1, _D_GAT))


def _ppass_body(as_hbm, ad_hbm, cv_hbm, src_hbm, dst_hbm, p_out,
                as_v, ad_v, cv_v, sidx_v, didx_v, p_v, *, nsb, n):
    cid = lax.axis_index("c")
    sid = lax.axis_index("s")
    wid = sid * _NCORE + cid
    pltpu.sync_copy(as_hbm, as_v)
    pltpu.sync_copy(ad_hbm, ad_v)
    pltpu.sync_copy(cv_hbm, cv_v)
    cv = cv_v[...]
    nmy = nsb // _NW + jnp.where(wid < nsb % _NW, 1, 0)

    @pl.loop(0, nmy)
    def _(i):
        s = i * _NW + wid
        pltpu.sync_copy(src_hbm.at[s], sidx_v)
        pltpu.sync_copy(dst_hbm.at[s], didx_v)
        for jj in range(_SB * _GRP // 16):
            g, off = jj // (_GRP // 16), jj % (_GRP // 16) * 16
            si = sidx_v[g, pl.ds(off, 16)]
            di = didx_v[g, pl.ds(off, 16)]
            a = plsc.load_gather(as_v, [si]) + plsc.load_gather(ad_v, [di])
            e = jnp.maximum(a, 0.2 * a)
            p_v[g, pl.ds(off, 16)] = jnp.exp(e - cv)
        pltpu.sync_copy(p_v, p_out.at[s])


def _ppass(a_s, a_d, cvec, src3d, dst3d):
    n = a_s.shape[0]
    nsb = src3d.shape[0]
    mesh = plsc.VectorSubcoreMesh(core_axis_name="c", subcore_axis_name="s")
    return pl.kernel(
        functools.partial(_ppass_body, nsb=nsb, n=n),
        out_type=jax.ShapeDtypeStruct((nsb, _SB, _GRP), _F32),
        mesh=mesh,
        scratch_types=[
            pltpu.VMEM((n,), _F32),
            pltpu.VMEM((n,), _F32),
            pltpu.VMEM((16,), _F32),
            pltpu.VMEM((_SB, _GRP), _I32),
            pltpu.VMEM((_SB, _GRP), _I32),
            pltpu.VMEM((_SB, _GRP), _F32),
        ],
        compiler_params=_sc_params(),
    )(a_s, a_d, cvec, src3d, dst3d)


def _gat_body(h_hbm, p_hbm, src_hbm, dst_hbm, z_hbm, out_hbm,
              sidx_v, didx_v, p_v, rows_v, s_sh,
              sem_g0, sem_g1, sem_s0, sem_s1,
              *, nsb, n):
    cid = lax.axis_index("c")
    sid = lax.axis_index("s")
    wid = sid * _NCORE + cid
    _init_shared(z_hbm, s_sh, sid, n)
    plsc.subcore_barrier()
    nmy = nsb // _NW + jnp.where(wid < nsb % _NW, 1, 0)

    @pl.loop(0, nmy)
    def _(i):
        s = i * _NW + wid
        pltpu.sync_copy(src_hbm.at[s], sidx_v)
        pltpu.sync_copy(dst_hbm.at[s], didx_v)
        pltpu.sync_copy(p_hbm.at[s], p_v)
        sem_g = (sem_g0, sem_g1)
        sem_s = (sem_s0, sem_s1)
        gds = {0: pltpu.async_copy(h_hbm.at[sidx_v.at[0]], rows_v.at[0],
                                   sem_g[0])}
        sds = {}
        for g in range(_SB):
            b = g % _NBUF
            gds[g].wait()

            @pl.loop(0, _GRP, unroll=4)
            def _(j):
                pj = plsc.load_gather(
                    p_v, [jnp.full((16,), g, _I32),
                          jnp.broadcast_to(j, (16,))])
                for k in range(_AWG // 16):
                    sl = (b, j, pl.ds(k * 16, 16))
                    rows_v[sl] = rows_v[sl] * pj

            if g + 1 < _SB:
                nb = (g + 1) % _NBUF
                if nb in sds:
                    sds[nb].wait()
                gds[g + 1] = pltpu.async_copy(h_hbm.at[sidx_v.at[g + 1]],
                                              rows_v.at[nb], sem_g[nb])
            sds[b] = pltpu.async_copy(rows_v.at[b], s_sh.at[didx_v.at[g]],
                                      sem_s[b], add=True)
        for b in sds:
            sds[b].wait()

    plsc.subcore_barrier()
    _dump_shared(s_sh, out_hbm, cid, sid, n)


def _gat_pass(h128, p3d, src3d, dst3d):
    n = h128.shape[0]
    nsb = src3d.shape[0]
    mesh = plsc.VectorSubcoreMesh(core_axis_name="c", subcore_axis_name="s")
    return pl.kernel(
        functools.partial(_gat_body, nsb=nsb, n=n),
        out_type=jax.ShapeDtypeStruct((_NCORE, n, _W), _F32),
        mesh=mesh,
        scratch_types=[
            pltpu.VMEM((_SB, _GRP), _I32),
            pltpu.VMEM((_SB, _GRP), _I32),
            pltpu.VMEM((_SB, _GRP), _F32),
            pltpu.VMEM((_NBUF, _GRP, _W), _F32),
            pltpu.VMEM_SHARED((n, _W), _F32),
            pltpu.SemaphoreType.DMA,
            pltpu.SemaphoreType.DMA,
            pltpu.SemaphoreType.DMA,
            pltpu.SemaphoreType.DMA,
        ],
        compiler_params=_sc_params(),
    )(h128, p3d, src3d, dst3d, jnp.zeros((n, _W), _F32))


def _combine_body(s2_ref, h_ref, ps_ref, bg_ref, wrg_ref, x1_ref, y_ref):
    n = h_ref.shape[0]
    ssum = s2_ref[0] + s2_ref[1]
    ps = ps_ref[...]
    h = h_ref[:, :_D_GAT]
    num = ssum[:, :_D_GAT] + ps * h
    denom = ssum[:, _D_GAT:_D_GAT + 1] + ps
    gat = num / denom + bg_ref[...]
    x1 = jnp.maximum(gat, 0.0)
    x1_ref[...] = x1
    y = jnp.dot(x1, wrg_ref[...], preferred_element_type=_F32)
    y_ref[...] = jnp.concatenate(
        [y, jnp.ones((n, 1), _F32), jnp.zeros((n, _W - _D_RG - 1), _F32)],
        axis=1)


def _combine(s2, h128, p_self, b_gat, w_rgcn):
    n = h128.shape[0]
    return pl.pallas_call(
        _combine_body,
        out_shape=[
            jax.ShapeDtypeStruct((n, _D_GAT), _F32),
            jax.ShapeDtypeStruct((n, _W), _F32),
        ],
    )(s2, h128, p_self, b_gat.reshape(1, _D_GAT), w_rgcn)


def _rgcn_body(y_hbm, src_hbm, dst_hbm, z_hbm, out_hbm,
               sidx_v, didx_v, rows_v, s_sh,
               sem_g0, sem_g1, sem_s0, sem_s1,
               *, nsb, n):
    cid = lax.axis_index("c")
    sid = lax.axis_index("s")
    wid = sid * _NCORE + cid
    _init_shared(z_hbm, s_sh, sid, n)
    plsc.subcore_barrier()
    nmy = nsb // _NW + jnp.where(wid < nsb % _NW, 1, 0)

    @pl.loop(0, nmy)
    def _(i):
        s = i * _NW + wid
        pltpu.sync_copy(src_hbm.at[s], sidx_v)
        pltpu.sync_copy(dst_hbm.at[s], didx_v)
        sem_g = (sem_g0, sem_g1)
        sem_s = (sem_s0, sem_s1)
        gds = {0: pltpu.async_copy(y_hbm.at[sidx_v.at[0]], rows_v.at[0],
                                   sem_g[0])}
        sds = {}
        for g in range(_SB):
            b = g % _NBUF
            gds[g].wait()
            if g + 1 < _SB:
                nb = (g + 1) % _NBUF
                if nb in sds:
                    sds[nb].wait()
                gds[g + 1] = pltpu.async_copy(y_hbm.at[sidx_v.at[g + 1]],
                                              rows_v.at[nb], sem_g[nb])
            sds[b] = pltpu.async_copy(rows_v.at[b], s_sh.at[didx_v.at[g]],
                                      sem_s[b], add=True)
        for b in sds:
            sds[b].wait()

    plsc.subcore_barrier()
    _dump_shared(s_sh, out_hbm, cid, sid, n)


def _rgcn_pass(y128, src3d, dst3d):
    n = y128.shape[0]
    nsb = src3d.shape[0]
    mesh = plsc.VectorSubcoreMesh(core_axis_name="c", subcore_axis_name="s")
    return pl.kernel(
        functools.partial(_rgcn_body, nsb=nsb, n=n),
        out_type=jax.ShapeDtypeStruct((_NCORE, n, _W), _F32),
        mesh=mesh,
        scratch_types=[
            pltpu.VMEM((_SB, _GRP), _I32),
            pltpu.VMEM((_SB, _GRP), _I32),
            pltpu.VMEM((_NBUF, _GRP, _W), _F32),
            pltpu.VMEM_SHARED((n, _W), _F32),
            pltpu.SemaphoreType.DMA,
            pltpu.SemaphoreType.DMA,
            pltpu.SemaphoreType.DMA,
            pltpu.SemaphoreType.DMA,
        ],
        compiler_params=_sc_params(),
    )(y128, src3d, dst3d, jnp.zeros((n, _W), _F32))


def _final_body(a2_ref, x1_ref, wroot_ref, brg_ref, wfc_ref, bfc_ref,
                out_ref):
    asum = a2_ref[0] + a2_ref[1]
    cnt = jnp.maximum(asum[:, _D_RG:_D_RG + 1], 1.0)
    agg = asum[:, :_D_RG] / cnt
    x2 = (agg + jnp.dot(x1_ref[...], wroot_ref[...],
                        preferred_element_type=_F32) + brg_ref[...])
    out_ref[...] = (jnp.dot(x2, wfc_ref[...], preferred_element_type=_F32)
                    + bfc_ref[...])


def _final(a2, x1, w_root, b_rgcn, w_fc, b_fc):
    n = x1.shape[0]
    return pl.pallas_call(
        _final_body,
        out_shape=jax.ShapeDtypeStruct((n, w_fc.shape[1]), _F32),
    )(a2, x1, w_root, b_rgcn.reshape(1, _D_RG), w_fc,
      b_fc.reshape(1, w_fc.shape[1]))


def kernel(x, edge_index, W_gat, att_src, att_dst, b_gat, W_rgcn, W_root,
           b_rgcn, W_fc, b_fc):
    n = x.shape[0]
    n_edges = edge_index.shape[1]
    nsb = n_edges // (_SB * _GRP)
    src = edge_index[0].astype(_I32)
    dst = edge_index[1].astype(_I32)
    src3d = src.reshape(nsb, _SB, _GRP)
    dst3d = dst.reshape(nsb, _SB, _GRP)

    h128, a_s, a_d, cvec, p_self = _prep(x, W_gat, att_src, att_dst)
    p3d = _ppass(a_s.reshape(n), a_d.reshape(n), cvec.reshape(16),
                 src3d, dst3d)
    s2 = _gat_pass(h128, p3d, src3d, dst3d)
    x1, y128 = _combine(s2, h128, p_self, b_gat, W_rgcn)
    a2 = _rgcn_pass(y128, src3d, dst3d)
    return _final(a2, x1, W_root, b_rgcn, W_fc, b_fc)


# speedup vs baseline: 1.2211x; 1.1185x over previous
import dataclasses
import functools

import jax
import jax.numpy as jnp
from jax import lax
from jax.experimental import pallas as pl
from jax.experimental.pallas import tpu as pltpu
from jax.experimental.pallas import tpu_sc as plsc

_F32 = jnp.float32
_I32 = jnp.int32
_GRP = 128
_SB = 10
_NBUF = 2
_NSUB = 16
_NCORE = 2
_NW = _NSUB * _NCORE
_D_GAT = 64
_D_RG = 32
_W = 128
_AWG = 80
_AWR = 48


def _sc_params():
    cp = pltpu.CompilerParams()
    if "needs_layout_passes" in pltpu.CompilerParams.__dataclass_fields__:
        cp = dataclasses.replace(cp, needs_layout_passes=False)
    return cp


def _row_split(n):
    per = ((n + _NSUB - 1) // _NSUB + 7) // 8 * 8
    last = n - per * (_NSUB - 1)
    assert last > 0 and last % 8 == 0
    return per, last


def _init_shared(z_hbm, s_sh, sid, n):
    per, last = _row_split(n)

    @pl.when(sid < _NSUB - 1)
    def _():
        pltpu.sync_copy(z_hbm.at[pl.ds(sid * per, per)],
                        s_sh.at[pl.ds(sid * per, per)])

    @pl.when(sid == _NSUB - 1)
    def _():
        pltpu.sync_copy(z_hbm.at[pl.ds(n - last, last)],
                        s_sh.at[pl.ds(n - last, last)])


def _dump_shared(s_sh, out_hbm, cid, sid, n):
    per, last = _row_split(n)

    @pl.when(sid < _NSUB - 1)
    def _():
        pltpu.sync_copy(s_sh.at[pl.ds(sid * per, per)],
                        out_hbm.at[cid, pl.ds(sid * per, per)])

    @pl.when(sid == _NSUB - 1)
    def _():
        pltpu.sync_copy(s_sh.at[pl.ds(n - last, last)],
                        out_hbm.at[cid, pl.ds(n - last, last)])


def _prep_body(x_ref, wg_ref, asw_ref, adw_ref,
               h_ref, as_ref, ad_ref, cv_ref, ps_ref):
    n = x_ref.shape[0]
    h = jnp.dot(x_ref[...], wg_ref[...], preferred_element_type=_F32)
    a_s = jnp.sum(h * asw_ref[...], axis=1, keepdims=True)
    a_d = jnp.sum(h * adw_ref[...], axis=1, keepdims=True)
    as_ref[...] = a_s
    ad_ref[...] = a_d
    cmax = jnp.max(a_s) + jnp.max(a_d)
    c = jnp.maximum(cmax, 0.2 * cmax)
    cv_ref[...] = jnp.full((1, 16), c, _F32)
    s = a_s + a_d
    e = jnp.maximum(s, 0.2 * s)
    ps_ref[...] = jnp.exp(e - c)
    h_ref[...] = jnp.concatenate(
        [h, jnp.ones((n, 1), _F32), jnp.zeros((n, _W - _D_GAT - 1), _F32)],
        axis=1)


def _prep(x, w_gat, att_src, att_dst):
    n = x.shape[0]
    return pl.pallas_call(
        _prep_body,
        out_shape=[
            jax.ShapeDtypeStruct((n, _W), _F32),
            jax.ShapeDtypeStruct((n, 1), _F32),
            jax.ShapeDtypeStruct((n, 1), _F32),
            jax.ShapeDtypeStruct((1, 16), _F32),
            jax.ShapeDtypeStruct((n, 1), _F32),
        ],
    )(x, w_gat, att_src.reshape(1, _D_GAT), att_dst.reshape(1, _D_GAT))


def _ppass_body(as_hbm, ad_hbm, cv_hbm, src_hbm, dst_hbm, p_out,
                as_v, ad_v, cv_v, sidx_v, didx_v, p_v, *, nsb, n):
    cid = lax.axis_index("c")
    sid = lax.axis_index("s")
    wid = sid * _NCORE + cid
    pltpu.sync_copy(as_hbm, as_v)
    pltpu.sync_copy(ad_hbm, ad_v)
    pltpu.sync_copy(cv_hbm, cv_v)
    cv = cv_v[...]
    nmy = nsb // _NW + jnp.where(wid < nsb % _NW, 1, 0)

    @pl.loop(0, nmy)
    def _(i):
        s = i * _NW + wid
        pltpu.sync_copy(src_hbm.at[s], sidx_v)
        pltpu.sync_copy(dst_hbm.at[s], didx_v)
        for jj in range(_SB * _GRP // 16):
            g, off = jj // (_GRP // 16), jj % (_GRP // 16) * 16
            si = sidx_v[g, pl.ds(off, 16)]
            di = didx_v[g, pl.ds(off, 16)]
            a = plsc.load_gather(as_v, [si]) + plsc.load_gather(ad_v, [di])
            e = jnp.maximum(a, 0.2 * a)
            p_v[g, pl.ds(off, 16)] = jnp.exp(e - cv)
        pltpu.sync_copy(p_v, p_out.at[s])


def _ppass(a_s, a_d, cvec, src3d, dst3d):
    n = a_s.shape[0]
    nsb = src3d.shape[0]
    mesh = plsc.VectorSubcoreMesh(core_axis_name="c", subcore_axis_name="s")
    return pl.kernel(
        functools.partial(_ppass_body, nsb=nsb, n=n),
        out_type=jax.ShapeDtypeStruct((nsb, _SB, _GRP), _F32),
        mesh=mesh,
        scratch_types=[
            pltpu.VMEM((n,), _F32),
            pltpu.VMEM((n,), _F32),
            pltpu.VMEM((16,), _F32),
            pltpu.VMEM((_SB, _GRP), _I32),
            pltpu.VMEM((_SB, _GRP), _I32),
            pltpu.VMEM((_SB, _GRP), _F32),
        ],
        compiler_params=_sc_params(),
    )(a_s, a_d, cvec, src3d, dst3d)


def _gat_body(h_hbm, p_hbm, src_hbm, dst_hbm, z_hbm, out_hbm,
              sidx_v, didx_v, p_v, rows_v, s_sh,
              sem_g0, sem_g1, sem_s0, sem_s1,
              *, nsb, n):
    cid = lax.axis_index("c")
    sid = lax.axis_index("s")
    wid = sid * _NCORE + cid
    _init_shared(z_hbm, s_sh, sid, n)
    plsc.subcore_barrier()
    nmy = nsb // _NW + jnp.where(wid < nsb % _NW, 1, 0)

    @pl.loop(0, nmy)
    def _(i):
        s = i * _NW + wid
        pltpu.sync_copy(src_hbm.at[s], sidx_v)
        pltpu.sync_copy(dst_hbm.at[s], didx_v)
        pltpu.sync_copy(p_hbm.at[s], p_v)
        sem_g = (sem_g0, sem_g1)
        sem_s = (sem_s0, sem_s1)
        gds = {0: pltpu.async_copy(h_hbm.at[sidx_v.at[0]], rows_v.at[0],
                                   sem_g[0])}
        sds = {}
        for g in range(_SB):
            b = g % _NBUF
            gds[g].wait()
            if g + 1 < _SB:
                nb = (g + 1) % _NBUF
                if nb in sds:
                    sds[nb].wait()
                gds[g + 1] = pltpu.async_copy(h_hbm.at[sidx_v.at[g + 1]],
                                              rows_v.at[nb], sem_g[nb])

            @pl.loop(0, _GRP, unroll=4)
            def _(j):
                pj = plsc.load_gather(
                    p_v, [jnp.full((16,), g, _I32),
                          jnp.broadcast_to(j, (16,))])
                for k in range(_AWG // 16):
                    sl = (b, j, pl.ds(k * 16, 16))
                    rows_v[sl] = rows_v[sl] * pj

            sds[b] = pltpu.async_copy(rows_v.at[b], s_sh.at[didx_v.at[g]],
                                      sem_s[b], add=True)
        for b in sds:
            sds[b].wait()

    plsc.subcore_barrier()
    _dump_shared(s_sh, out_hbm, cid, sid, n)


def _gat_pass(h128, p3d, src3d, dst3d):
    n = h128.shape[0]
    nsb = src3d.shape[0]
    mesh = plsc.VectorSubcoreMesh(core_axis_name="c", subcore_axis_name="s")
    return pl.kernel(
        functools.partial(_gat_body, nsb=nsb, n=n),
        out_type=jax.ShapeDtypeStruct((_NCORE, n, _W), _F32),
        mesh=mesh,
        scratch_types=[
            pltpu.VMEM((_SB, _GRP), _I32),
            pltpu.VMEM((_SB, _GRP), _I32),
            pltpu.VMEM((_SB, _GRP), _F32),
            pltpu.VMEM((_NBUF, _GRP, _W), _F32),
            pltpu.VMEM_SHARED((n, _W), _F32),
            pltpu.SemaphoreType.DMA,
            pltpu.SemaphoreType.DMA,
            pltpu.SemaphoreType.DMA,
            pltpu.SemaphoreType.DMA,
        ],
        compiler_params=_sc_params(),
    )(h128, p3d, src3d, dst3d, jnp.zeros((n, _W), _F32))


def _combine_body(s2_ref, h_ref, ps_ref, bg_ref, wrg_ref, x1_ref, y_ref):
    n = h_ref.shape[0]
    ssum = s2_ref[0] + s2_ref[1]
    ps = ps_ref[...]
    h = h_ref[:, :_D_GAT]
    num = ssum[:, :_D_GAT] + ps * h
    denom = ssum[:, _D_GAT:_D_GAT + 1] + ps
    gat = num / denom + bg_ref[...]
    x1 = jnp.maximum(gat, 0.0)
    x1_ref[...] = x1
    y = jnp.dot(x1, wrg_ref[...], preferred_element_type=_F32)
    y_ref[...] = jnp.concatenate(
        [y, jnp.ones((n, 1), _F32), jnp.zeros((n, _W - _D_RG - 1), _F32)],
        axis=1)


def _combine(s2, h128, p_self, b_gat, w_rgcn):
    n = h128.shape[0]
    return pl.pallas_call(
        _combine_body,
        out_shape=[
            jax.ShapeDtypeStruct((n, _D_GAT), _F32),
            jax.ShapeDtypeStruct((n, _W), _F32),
        ],
    )(s2, h128, p_self, b_gat.reshape(1, _D_GAT), w_rgcn)


def _rgcn_body(y_hbm, src_hbm, dst_hbm, z_hbm, out_hbm,
               sidx_v, didx_v, rows_v, s_sh,
               sem_g0, sem_g1, sem_s0, sem_s1,
               *, nsb, n):
    cid = lax.axis_index("c")
    sid = lax.axis_index("s")
    wid = sid * _NCORE + cid
    _init_shared(z_hbm, s_sh, sid, n)
    plsc.subcore_barrier()
    nmy = nsb // _NW + jnp.where(wid < nsb % _NW, 1, 0)

    @pl.loop(0, nmy)
    def _(i):
        s = i * _NW + wid
        pltpu.sync_copy(src_hbm.at[s], sidx_v)
        pltpu.sync_copy(dst_hbm.at[s], didx_v)
        sem_g = (sem_g0, sem_g1)
        sem_s = (sem_s0, sem_s1)
        gds = {0: pltpu.async_copy(y_hbm.at[sidx_v.at[0]], rows_v.at[0],
                                   sem_g[0])}
        sds = {}
        for g in range(_SB):
            b = g % _NBUF
            gds[g].wait()
            if g + 1 < _SB:
                nb = (g + 1) % _NBUF
                if nb in sds:
                    sds[nb].wait()
                gds[g + 1] = pltpu.async_copy(y_hbm.at[sidx_v.at[g + 1]],
                                              rows_v.at[nb], sem_g[nb])
            sds[b] = pltpu.async_copy(rows_v.at[b], s_sh.at[didx_v.at[g]],
                                      sem_s[b], add=True)
        for b in sds:
            sds[b].wait()

    plsc.subcore_barrier()
    _dump_shared(s_sh, out_hbm, cid, sid, n)


def _rgcn_pass(y128, src3d, dst3d):
    n = y128.shape[0]
    nsb = src3d.shape[0]
    mesh = plsc.VectorSubcoreMesh(core_axis_name="c", subcore_axis_name="s")
    return pl.kernel(
        functools.partial(_rgcn_body, nsb=nsb, n=n),
        out_type=jax.ShapeDtypeStruct((_NCORE, n, _W), _F32),
        mesh=mesh,
        scratch_types=[
            pltpu.VMEM((_SB, _GRP), _I32),
            pltpu.VMEM((_SB, _GRP), _I32),
            pltpu.VMEM((_NBUF, _GRP, _W), _F32),
            pltpu.VMEM_SHARED((n, _W), _F32),
            pltpu.SemaphoreType.DMA,
            pltpu.SemaphoreType.DMA,
            pltpu.SemaphoreType.DMA,
            pltpu.SemaphoreType.DMA,
        ],
        compiler_params=_sc_params(),
    )(y128, src3d, dst3d, jnp.zeros((n, _W), _F32))


def _final_body(a2_ref, x1_ref, wroot_ref, brg_ref, wfc_ref, bfc_ref,
                out_ref):
    asum = a2_ref[0] + a2_ref[1]
    cnt = jnp.maximum(asum[:, _D_RG:_D_RG + 1], 1.0)
    agg = asum[:, :_D_RG] / cnt
    x2 = (agg + jnp.dot(x1_ref[...], wroot_ref[...],
                        preferred_element_type=_F32) + brg_ref[...])
    out_ref[...] = (jnp.dot(x2, wfc_ref[...], preferred_element_type=_F32)
                    + bfc_ref[...])


def _final(a2, x1, w_root, b_rgcn, w_fc, b_fc):
    n = x1.shape[0]
    return pl.pallas_call(
        _final_body,
        out_shape=jax.ShapeDtypeStruct((n, w_fc.shape[1]), _F32),
    )(a2, x1, w_root, b_rgcn.reshape(1, _D_RG), w_fc,
      b_fc.reshape(1, w_fc.shape[1]))


def kernel(x, edge_index, W_gat, att_src, att_dst, b_gat, W_rgcn, W_root,
           b_rgcn, W_fc, b_fc):
    n = x.shape[0]
    n_edges = edge_index.shape[1]
    nsb = n_edges // (_SB * _GRP)
    src = edge_index[0].astype(_I32)
    dst = edge_index[1].astype(_I32)
    src3d = src.reshape(nsb, _SB, _GRP)
    dst3d = dst.reshape(nsb, _SB, _GRP)

    h128, a_s, a_d, cvec, p_self = _prep(x, W_gat, att_src, att_dst)
    p3d = _ppass(a_s.reshape(n), a_d.reshape(n), cvec.reshape(16),
                 src3d, dst3d)
    s2 = _gat_pass(h128, p3d, src3d, dst3d)
    x1, y128 = _combine(s2, h128, p_self, b_gat, W_rgcn)
    a2 = _rgcn_pass(y128, src3d, dst3d)
    return _final(a2, x1, W_root, b_rgcn, W_fc, b_fc)
